# Initial kernel scaffold; baseline (speedup 1.0000x reference)
import jax
import jax.numpy as jnp
from jax.experimental import pallas as pl


def kernel(x_author, x_paper, edge_index, supervision_edge_index, W_paper, b_paper, W_author, b_author):
    raise NotImplementedError("write your pallas kernel here")


# speedup vs baseline: 2.6073x; 2.6073x over previous
import functools

import jax
import jax.numpy as jnp
from jax import lax
from jax.experimental import pallas as pl
from jax.experimental.pallas import tpu as pltpu
from jax.experimental.pallas import tpu_sc as plsc


def _sc_geometry():
    try:
        info = plsc.get_sparse_core_info()
        return int(info.num_cores), int(info.num_subcores)
    except Exception:
        return 2, 16


def _chunk_size(total, limit=128):
    for g in range(limit, 7, -8):
        if total % g == 0:
            return g
    raise ValueError(f"no valid chunk size for {total}")


def _paper_linear(x_paper, W_paper, b_paper2d):
    NP, PIN = x_paper.shape
    H = W_paper.shape[0]
    HH = H // 2
    BLK = 2000

    def body(x_ref, w_ref, b_ref, full_ref, h0_ref, h1_ref):
        h = lax.dot_general(x_ref[...], w_ref[...], (((1,), (1,)), ((), ())),
                            preferred_element_type=jnp.float32)
        h = h + b_ref[...]
        full_ref[...] = h
        h0_ref[...] = h[:, :HH]
        h1_ref[...] = h[:, HH:]

    return pl.pallas_call(
        body,
        grid=(NP // BLK,),
        in_specs=[
            pl.BlockSpec((BLK, PIN), lambda i: (i, 0)),
            pl.BlockSpec((H, PIN), lambda i: (0, 0)),
            pl.BlockSpec((1, H), lambda i: (0, 0)),
        ],
        out_specs=[
            pl.BlockSpec((BLK, H), lambda i: (i, 0)),
            pl.BlockSpec((BLK, HH), lambda i: (i, 0)),
            pl.BlockSpec((BLK, HH), lambda i: (i, 0)),
        ],
        out_shape=[
            jax.ShapeDtypeStruct((NP, H), jnp.float32),
            jax.ShapeDtypeStruct((NP, HH), jnp.float32),
            jax.ShapeDtypeStruct((NP, HH), jnp.float32),
        ],
    )(x_paper, W_paper, b_paper2d)


def _segment_sum_sc(ph0, ph1, e_author, e_paper, n_authors):
    NE = e_author.shape[0]
    HH = ph0.shape[1]
    NC, NS = _sc_geometry()
    E_PER = NE // NS
    assert E_PER * NS == NE
    G = _chunk_size(E_PER)
    n_chunks = E_PER // G
    ZR = G
    ROWS_PER_TILE = -(-n_authors // NS)
    ROWS_PER_TILE = -(-ROWS_PER_TILE // ZR) * ZR
    na_pad = ROWS_PER_TILE * NS
    n_zc = ROWS_PER_TILE // ZR

    mesh = plsc.VectorSubcoreMesh(core_axis_name="c", subcore_axis_name="s")

    @functools.partial(
        pl.kernel,
        out_type=[
            jax.ShapeDtypeStruct((na_pad, HH), jnp.float32),
            jax.ShapeDtypeStruct((na_pad, HH), jnp.float32),
            jax.ShapeDtypeStruct((NS * na_pad,), jnp.float32),
        ],
        mesh=mesh,
        compiler_params=pltpu.CompilerParams(needs_layout_passes=False),
        scratch_types=[
            pltpu.VMEM((G,), jnp.int32),
            pltpu.VMEM((G,), jnp.int32),
            pltpu.VMEM((G, HH), jnp.float32),
            pltpu.VMEM((na_pad,), jnp.float32),
            pltpu.VMEM_SHARED((na_pad, HH), jnp.float32),
            pltpu.SemaphoreType.DMA,
        ],
    )
    def kern(ph0_hbm, ph1_hbm, ea_hbm, ep_hbm, zrow_hbm, zhist_hbm,
             s0_hbm, s1_hbm, cnt_hbm,
             aidx_v, pidx_v, rows_v, hist_v, acc_sh, sem):
        cid = lax.axis_index("c")
        sid = lax.axis_index("s")
        one16 = jnp.ones((16,), jnp.float32)

        pltpu.sync_copy(zrow_hbm, rows_v)
        pltpu.sync_copy(zhist_hbm, hist_v)
        row_lo = sid * ROWS_PER_TILE
        for k in range(n_zc):
            pltpu.sync_copy(rows_v, acc_sh.at[pl.ds(row_lo + k * ZR, ZR)])
        plsc.subcore_barrier()

        edge_lo = sid * E_PER

        def run_accum(ph_hbm, do_hist):
            def chunk(g, _):
                base = edge_lo + g * G
                pltpu.sync_copy(ea_hbm.at[pl.ds(base, G)], aidx_v)
                pltpu.sync_copy(ep_hbm.at[pl.ds(base, G)], pidx_v)
                pltpu.async_copy(ph_hbm.at[pidx_v], rows_v, sem).wait()
                pltpu.sync_copy(rows_v, acc_sh.at[aidx_v], add=True)
                if do_hist:
                    for i in range(G // 16):
                        av = aidx_v[pl.ds(i * 16, 16)]
                        plsc.addupdate_scatter(hist_v, [av], one16)
                return 0
            lax.fori_loop(0, n_chunks, chunk, 0)

        @pl.when(cid == 0)
        def _():
            run_accum(ph0_hbm, True)

        @pl.when(cid == 1)
        def _():
            run_accum(ph1_hbm, False)

        plsc.subcore_barrier()

        def copy_out(dst_hbm):
            for k in range(n_zc):
                sl = pl.ds(row_lo + k * ZR, ZR)
                pltpu.sync_copy(acc_sh.at[sl], rows_v)
                pltpu.sync_copy(rows_v, dst_hbm.at[sl])

        @pl.when(cid == 0)
        def _():
            copy_out(s0_hbm)
            pltpu.sync_copy(hist_v, cnt_hbm.at[pl.ds(sid * na_pad, na_pad)])

        @pl.when(cid == 1)
        def _():
            copy_out(s1_hbm)

    zrow = jnp.zeros((ZR, HH), jnp.float32)
    zhist = jnp.zeros((na_pad,), jnp.float32)
    s0, s1, cnt_flat = kern(ph0, ph1, e_author, e_paper, zrow, zhist)
    return s0, s1, cnt_flat.reshape(NS, na_pad)


def _author_linear(s0, s1, cnt, W_author, b_author2d):
    NA, HH = s0.shape
    H = W_author.shape[0]
    NT = cnt.shape[0]
    BLK = next(b for b in range(2048, 7, -8) if NA % b == 0)

    def body(s0_ref, s1_ref, c_ref, w_ref, b_ref, out_ref):
        w = w_ref[...]
        h = lax.dot_general(s0_ref[...], w[:, :HH], (((1,), (1,)), ((), ())),
                            preferred_element_type=jnp.float32)
        h = h + lax.dot_general(s1_ref[...], w[:, HH:], (((1,), (1,)), ((), ())),
                                preferred_element_type=jnp.float32)
        counts = jnp.sum(c_ref[...], axis=0)[:, None]
        scale = 1.0 / jnp.maximum(counts, 1.0)
        out_ref[...] = h * scale + b_ref[...]

    return pl.pallas_call(
        body,
        grid=(NA // BLK,),
        in_specs=[
            pl.BlockSpec((BLK, HH), lambda i: (i, 0)),
            pl.BlockSpec((BLK, HH), lambda i: (i, 0)),
            pl.BlockSpec((NT, BLK), lambda i: (0, i)),
            pl.BlockSpec((H, H), lambda i: (0, 0)),
            pl.BlockSpec((1, H), lambda i: (0, 0)),
        ],
        out_specs=pl.BlockSpec((BLK, H), lambda i: (i, 0)),
        out_shape=jax.ShapeDtypeStruct((NA, H), jnp.float32),
    )(s0, s1, cnt, W_author, b_author2d)


def _score_sc(author_h, paper_h, s_author, s_paper):
    NSUP = s_author.shape[0]
    H = author_h.shape[1]
    NC, NS = _sc_geometry()
    NW = NC * NS
    S_PER = NSUP // NW
    assert S_PER * NW == NSUP
    G = _chunk_size(S_PER)
    n_chunks = S_PER // G
    GP = ((G + 15) // 16) * 16
    n_groups = GP // 16

    mesh = plsc.VectorSubcoreMesh(core_axis_name="c", subcore_axis_name="s")

    @functools.partial(
        pl.kernel,
        out_type=jax.ShapeDtypeStruct((NSUP,), jnp.float32),
        mesh=mesh,
        compiler_params=pltpu.CompilerParams(needs_layout_passes=False),
        scratch_types=[
            pltpu.VMEM((G,), jnp.int32),
            pltpu.VMEM((G,), jnp.int32),
            pltpu.VMEM((G, H), jnp.float32),
            pltpu.VMEM((G, H), jnp.float32),
            pltpu.VMEM((GP * 16,), jnp.float32),
            pltpu.VMEM((GP,), jnp.float32),
            pltpu.SemaphoreType.DMA,
            pltpu.SemaphoreType.DMA,
        ],
    )
    def kern(ah_hbm, ph_hbm, sa_hbm, sp_hbm, out_hbm,
             aidx_v, pidx_v, arows_v, prows_v, part_v, outb_v, sem_a, sem_p):
        cid = lax.axis_index("c")
        sid = lax.axis_index("s")
        wid = sid * NC + cid
        base0 = wid * S_PER
        lane = lax.iota(jnp.int32, 16)

        def chunk(g, _):
            base = base0 + g * G
            pltpu.sync_copy(sa_hbm.at[pl.ds(base, G)], aidx_v)
            pltpu.sync_copy(sp_hbm.at[pl.ds(base, G)], pidx_v)
            cp_a = pltpu.async_copy(ah_hbm.at[aidx_v], arows_v, sem_a)
            cp_p = pltpu.async_copy(ph_hbm.at[pidx_v], prows_v, sem_p)
            cp_a.wait()
            cp_p.wait()

            def edge(e, _):
                acc = jnp.zeros((16,), jnp.float32)
                for c in range(H // 16):
                    sl = pl.ds(c * 16, 16)
                    acc = acc + arows_v[e, sl] * prows_v[e, sl]
                part_v[pl.ds(e * 16, 16)] = acc
                return 0
            lax.fori_loop(0, G, edge, 0)

            for grp in range(n_groups):
                rows = (jnp.full((16,), grp * 16, jnp.int32) + lane) * 16
                tot = jnp.zeros((16,), jnp.float32)
                for j in range(16):
                    tot = tot + plsc.load_gather(part_v, [rows + j])
                outb_v[pl.ds(grp * 16, 16)] = tot

            pltpu.sync_copy(outb_v.at[pl.ds(0, G)], out_hbm.at[pl.ds(base, G)])
            return 0

        lax.fori_loop(0, n_chunks, chunk, 0)

    return kern(author_h, paper_h, s_author, s_paper)


def kernel(x_author, x_paper, edge_index, supervision_edge_index,
           W_paper, b_paper, W_author, b_author):
    n_authors = x_author.shape[0]
    ei = edge_index.astype(jnp.int32)
    si = supervision_edge_index.astype(jnp.int32)
    e_author, e_paper = ei[0], ei[1]
    s_author, s_paper = si[0], si[1]

    paper_h, ph0, ph1 = _paper_linear(x_paper, W_paper,
                                      b_paper.reshape(1, -1))
    s0, s1, cnt = _segment_sum_sc(ph0, ph1, e_author, e_paper, n_authors)
    author_h = _author_linear(s0, s1, cnt, W_author, b_author.reshape(1, -1))
    return _score_sc(author_h, paper_h, s_author, s_paper)


# speedup vs baseline: 3.3668x; 1.2913x over previous
import functools

import jax
import jax.numpy as jnp
from jax import lax
from jax.experimental import pallas as pl
from jax.experimental.pallas import tpu as pltpu
from jax.experimental.pallas import tpu_sc as plsc


def _sc_geometry():
    try:
        info = plsc.get_sparse_core_info()
        return int(info.num_cores), int(info.num_subcores)
    except Exception:
        return 2, 16


def _chunk_size(total, limit=128):
    for g in range(limit, 7, -8):
        if total % g == 0:
            return g
    raise ValueError(f"no valid chunk size for {total}")


def _paper_linear(x_paper, W_paper, b_paper2d):
    NP, PIN = x_paper.shape
    H = W_paper.shape[0]
    HH = H // 2
    BLK = 2000

    def body(x_ref, w_ref, b_ref, full_ref, h0_ref, h1_ref):
        h = lax.dot_general(x_ref[...], w_ref[...], (((1,), (1,)), ((), ())),
                            preferred_element_type=jnp.float32)
        h = h + b_ref[...]
        full_ref[...] = h
        h0_ref[...] = h[:, :HH]
        h1_ref[...] = h[:, HH:]

    return pl.pallas_call(
        body,
        grid=(NP // BLK,),
        in_specs=[
            pl.BlockSpec((BLK, PIN), lambda i: (i, 0)),
            pl.BlockSpec((H, PIN), lambda i: (0, 0)),
            pl.BlockSpec((1, H), lambda i: (0, 0)),
        ],
        out_specs=[
            pl.BlockSpec((BLK, H), lambda i: (i, 0)),
            pl.BlockSpec((BLK, HH), lambda i: (i, 0)),
            pl.BlockSpec((BLK, HH), lambda i: (i, 0)),
        ],
        out_shape=[
            jax.ShapeDtypeStruct((NP, H), jnp.float32),
            jax.ShapeDtypeStruct((NP, HH), jnp.float32),
            jax.ShapeDtypeStruct((NP, HH), jnp.float32),
        ],
    )(x_paper, W_paper, b_paper2d)


def _segment_sum_sc(ph0, ph1, e_author, e_paper, n_authors):
    NE = e_author.shape[0]
    HH = ph0.shape[1]
    NC, NS = _sc_geometry()
    E_PER = NE // NS
    assert E_PER * NS == NE
    G = _chunk_size(E_PER)
    n_chunks = E_PER // G
    ZR = G
    ROWS_PER_TILE = -(-n_authors // NS)
    ROWS_PER_TILE = -(-ROWS_PER_TILE // ZR) * ZR
    na_pad = ROWS_PER_TILE * NS
    n_zc = ROWS_PER_TILE // ZR

    mesh = plsc.VectorSubcoreMesh(core_axis_name="c", subcore_axis_name="s")

    @functools.partial(
        pl.kernel,
        out_type=[
            jax.ShapeDtypeStruct((na_pad, HH), jnp.float32),
            jax.ShapeDtypeStruct((na_pad, HH), jnp.float32),
            jax.ShapeDtypeStruct((NS * na_pad,), jnp.float32),
        ],
        mesh=mesh,
        compiler_params=pltpu.CompilerParams(needs_layout_passes=False),
        scratch_types=[
            pltpu.VMEM((G,), jnp.int32),
            pltpu.VMEM((G,), jnp.int32),
            pltpu.VMEM((G,), jnp.int32),
            pltpu.VMEM((G,), jnp.int32),
            pltpu.VMEM((G, HH), jnp.float32),
            pltpu.VMEM((G, HH), jnp.float32),
            pltpu.VMEM((na_pad,), jnp.float32),
            pltpu.VMEM_SHARED((na_pad, HH), jnp.float32),
            pltpu.SemaphoreType.DMA,
            pltpu.SemaphoreType.DMA,
        ],
    )
    def kern(ph0_hbm, ph1_hbm, ea_hbm, ep_hbm, zrow_hbm, zhist_hbm,
             s0_hbm, s1_hbm, cnt_hbm,
             aidx0_v, aidx1_v, pidx0_v, pidx1_v, rows0_v, rows1_v,
             hist_v, acc_sh, sem0, sem1):
        cid = lax.axis_index("c")
        sid = lax.axis_index("s")
        one16 = jnp.ones((16,), jnp.float32)
        aidx = (aidx0_v, aidx1_v)
        pidx = (pidx0_v, pidx1_v)
        rows = (rows0_v, rows1_v)
        sems = (sem0, sem1)

        pltpu.sync_copy(zrow_hbm, rows0_v)
        pltpu.sync_copy(zhist_hbm, hist_v)
        row_lo = sid * ROWS_PER_TILE
        for k in range(n_zc):
            pltpu.sync_copy(rows0_v, acc_sh.at[pl.ds(row_lo + k * ZR, ZR)])
        plsc.subcore_barrier()

        edge_lo = sid * E_PER

        def run_accum(ph_hbm, do_hist):
            def fetch(g, b):
                base = edge_lo + g * G
                pltpu.sync_copy(ea_hbm.at[pl.ds(base, G)], aidx[b])
                pltpu.sync_copy(ep_hbm.at[pl.ds(base, G)], pidx[b])
                pltpu.async_copy(ph_hbm.at[pidx[b]], rows[b], sems[b])

            def process(g, b, prefetch):
                pltpu.make_async_copy(ph_hbm.at[pidx[b]], rows[b],
                                      sems[b]).wait()
                if prefetch:
                    fetch(g + 1, 1 - b)
                pltpu.sync_copy(rows[b], acc_sh.at[aidx[b]], add=True)
                if do_hist:
                    for i in range(G // 16):
                        av = aidx[b][pl.ds(i * 16, 16)]
                        plsc.addupdate_scatter(hist_v, [av], one16)

            fetch(0, 0)

            def pair(k, _):
                g = 2 * k
                process(g, 0, True)
                process(g + 1, 1, True)
                return 0
            n_pairs = (n_chunks - 1) // 2
            lax.fori_loop(0, n_pairs, pair, 0)
            for g in range(2 * n_pairs, n_chunks):
                process(g, g % 2, g < n_chunks - 1)

        @pl.when(cid == 0)
        def _():
            run_accum(ph0_hbm, True)

        @pl.when(cid == 1)
        def _():
            run_accum(ph1_hbm, False)

        plsc.subcore_barrier()

        def copy_out(dst_hbm):
            for k in range(n_zc):
                sl = pl.ds(row_lo + k * ZR, ZR)
                pltpu.sync_copy(acc_sh.at[sl], rows0_v)
                pltpu.sync_copy(rows0_v, dst_hbm.at[sl])

        @pl.when(cid == 0)
        def _():
            copy_out(s0_hbm)
            pltpu.sync_copy(hist_v, cnt_hbm.at[pl.ds(sid * na_pad, na_pad)])

        @pl.when(cid == 1)
        def _():
            copy_out(s1_hbm)

    zrow = jnp.zeros((ZR, HH), jnp.float32)
    zhist = jnp.zeros((na_pad,), jnp.float32)
    s0, s1, cnt_flat = kern(ph0, ph1, e_author, e_paper, zrow, zhist)
    return s0, s1, cnt_flat.reshape(NS, na_pad)


def _author_linear(s0, s1, cnt, W_author, b_author2d):
    NA, HH = s0.shape
    H = W_author.shape[0]
    NT = cnt.shape[0]
    BLK = next(b for b in range(2048, 7, -8) if NA % b == 0)

    def body(s0_ref, s1_ref, c_ref, w_ref, b_ref, out_ref):
        w = w_ref[...]
        h = lax.dot_general(s0_ref[...], w[:, :HH], (((1,), (1,)), ((), ())),
                            preferred_element_type=jnp.float32)
        h = h + lax.dot_general(s1_ref[...], w[:, HH:], (((1,), (1,)), ((), ())),
                                preferred_element_type=jnp.float32)
        counts = jnp.sum(c_ref[...], axis=0)[:, None]
        scale = 1.0 / jnp.maximum(counts, 1.0)
        out_ref[...] = h * scale + b_ref[...]

    return pl.pallas_call(
        body,
        grid=(NA // BLK,),
        in_specs=[
            pl.BlockSpec((BLK, HH), lambda i: (i, 0)),
            pl.BlockSpec((BLK, HH), lambda i: (i, 0)),
            pl.BlockSpec((NT, BLK), lambda i: (0, i)),
            pl.BlockSpec((H, H), lambda i: (0, 0)),
            pl.BlockSpec((1, H), lambda i: (0, 0)),
        ],
        out_specs=pl.BlockSpec((BLK, H), lambda i: (i, 0)),
        out_shape=jax.ShapeDtypeStruct((NA, H), jnp.float32),
    )(s0, s1, cnt, W_author, b_author2d)


def _score_sc(author_h, paper_h, s_author, s_paper):
    NSUP = s_author.shape[0]
    H = author_h.shape[1]
    NC, NS = _sc_geometry()
    NW = NC * NS
    S_PER = NSUP // NW
    assert S_PER * NW == NSUP
    G = _chunk_size(S_PER)
    n_chunks = S_PER // G
    GP = ((G + 15) // 16) * 16
    n_groups = GP // 16

    mesh = plsc.VectorSubcoreMesh(core_axis_name="c", subcore_axis_name="s")

    @functools.partial(
        pl.kernel,
        out_type=jax.ShapeDtypeStruct((NSUP,), jnp.float32),
        mesh=mesh,
        compiler_params=pltpu.CompilerParams(needs_layout_passes=False),
        scratch_types=[
            pltpu.VMEM((G,), jnp.int32),
            pltpu.VMEM((G,), jnp.int32),
            pltpu.VMEM((G,), jnp.int32),
            pltpu.VMEM((G,), jnp.int32),
            pltpu.VMEM((G, H), jnp.float32),
            pltpu.VMEM((G, H), jnp.float32),
            pltpu.VMEM((G, H), jnp.float32),
            pltpu.VMEM((G, H), jnp.float32),
            pltpu.VMEM((GP * 16,), jnp.float32),
            pltpu.VMEM((GP,), jnp.float32),
            pltpu.SemaphoreType.DMA,
            pltpu.SemaphoreType.DMA,
        ],
    )
    def kern(ah_hbm, ph_hbm, sa_hbm, sp_hbm, out_hbm,
             aidx0_v, aidx1_v, pidx0_v, pidx1_v,
             arows0_v, arows1_v, prows0_v, prows1_v,
             part_v, outb_v, sem0, sem1):
        cid = lax.axis_index("c")
        sid = lax.axis_index("s")
        wid = sid * NC + cid
        base0 = wid * S_PER
        lane = lax.iota(jnp.int32, 16)
        aidx = (aidx0_v, aidx1_v)
        pidx = (pidx0_v, pidx1_v)
        arows = (arows0_v, arows1_v)
        prows = (prows0_v, prows1_v)
        sems = (sem0, sem1)

        def fetch(g, b):
            base = base0 + g * G
            pltpu.sync_copy(sa_hbm.at[pl.ds(base, G)], aidx[b])
            pltpu.sync_copy(sp_hbm.at[pl.ds(base, G)], pidx[b])
            pltpu.async_copy(ah_hbm.at[aidx[b]], arows[b], sems[b])
            pltpu.async_copy(ph_hbm.at[pidx[b]], prows[b], sems[b])

        def process(g, b, prefetch):
            pltpu.make_async_copy(ah_hbm.at[aidx[b]], arows[b], sems[b]).wait()
            pltpu.make_async_copy(ph_hbm.at[pidx[b]], prows[b], sems[b]).wait()
            if prefetch:
                fetch(g + 1, 1 - b)

            def edge(e, _):
                acc = jnp.zeros((16,), jnp.float32)
                for c in range(H // 16):
                    sl = pl.ds(c * 16, 16)
                    acc = acc + arows[b][e, sl] * prows[b][e, sl]
                part_v[pl.ds(e * 16, 16)] = acc
                return 0
            lax.fori_loop(0, G, edge, 0)

            for grp in range(n_groups):
                rows = (jnp.full((16,), grp * 16, jnp.int32) + lane) * 16
                tot = jnp.zeros((16,), jnp.float32)
                for j in range(16):
                    tot = tot + plsc.load_gather(part_v, [rows + j])
                outb_v[pl.ds(grp * 16, 16)] = tot

            base = base0 + g * G
            pltpu.sync_copy(outb_v.at[pl.ds(0, G)], out_hbm.at[pl.ds(base, G)])

        fetch(0, 0)

        def pair(k, _):
            g = 2 * k
            process(g, 0, True)
            process(g + 1, 1, True)
            return 0
        n_pairs = (n_chunks - 1) // 2
        lax.fori_loop(0, n_pairs, pair, 0)
        for g in range(2 * n_pairs, n_chunks):
            process(g, g % 2, g < n_chunks - 1)

    return kern(author_h, paper_h, s_author, s_paper)


def kernel(x_author, x_paper, edge_index, supervision_edge_index,
           W_paper, b_paper, W_author, b_author):
    n_authors = x_author.shape[0]
    ei = edge_index.astype(jnp.int32)
    si = supervision_edge_index.astype(jnp.int32)
    e_author, e_paper = ei[0], ei[1]
    s_author, s_paper = si[0], si[1]

    paper_h, ph0, ph1 = _paper_linear(x_paper, W_paper,
                                      b_paper.reshape(1, -1))
    s0, s1, cnt = _segment_sum_sc(ph0, ph1, e_author, e_paper, n_authors)
    author_h = _author_linear(s0, s1, cnt, W_author, b_author.reshape(1, -1))
    return _score_sc(author_h, paper_h, s_author, s_paper)


# speedup vs baseline: 4.5339x; 1.3467x over previous
import functools

import jax
import jax.numpy as jnp
from jax import lax
from jax.experimental import pallas as pl
from jax.experimental.pallas import tpu as pltpu
from jax.experimental.pallas import tpu_sc as plsc


def _sc_geometry():
    try:
        info = plsc.get_sparse_core_info()
        return int(info.num_cores), int(info.num_subcores)
    except Exception:
        return 2, 16


def _chunk_size(total, limit=128):
    for g in range(limit, 7, -8):
        if total % g == 0:
            return g
    raise ValueError(f"no valid chunk size for {total}")


def _paper_linear(x_paper, W_paper, b_paper2d):
    NP, PIN = x_paper.shape
    H = W_paper.shape[0]
    HH = H // 2
    BLK = 2000

    def body(x_ref, w_ref, b_ref, full_ref, h0_ref, h1_ref):
        h = lax.dot_general(x_ref[...], w_ref[...], (((1,), (1,)), ((), ())),
                            preferred_element_type=jnp.float32)
        h = h + b_ref[...]
        full_ref[...] = h
        h0_ref[...] = h[:, :HH]
        h1_ref[...] = h[:, HH:]

    return pl.pallas_call(
        body,
        grid=(NP // BLK,),
        in_specs=[
            pl.BlockSpec((BLK, PIN), lambda i: (i, 0)),
            pl.BlockSpec((H, PIN), lambda i: (0, 0)),
            pl.BlockSpec((1, H), lambda i: (0, 0)),
        ],
        out_specs=[
            pl.BlockSpec((BLK, H), lambda i: (i, 0)),
            pl.BlockSpec((BLK, HH), lambda i: (i, 0)),
            pl.BlockSpec((BLK, HH), lambda i: (i, 0)),
        ],
        out_shape=[
            jax.ShapeDtypeStruct((NP, H), jnp.float32),
            jax.ShapeDtypeStruct((NP, HH), jnp.float32),
            jax.ShapeDtypeStruct((NP, HH), jnp.float32),
        ],
    )(x_paper, W_paper, b_paper2d)


def _segment_sum_sc(ph0, ph1, e_author, e_paper, n_authors):
    NE = e_author.shape[0]
    HH = ph0.shape[1]
    NC, NS = _sc_geometry()
    E_PER = NE // NS
    assert E_PER * NS == NE
    G = _chunk_size(E_PER)
    n_chunks = E_PER // G
    CPB = 3
    IB = CPB * G
    ZR = G
    ROWS_PER_TILE = -(-n_authors // NS)
    ROWS_PER_TILE = -(-ROWS_PER_TILE // ZR) * ZR
    na_pad = ROWS_PER_TILE * NS
    n_zc = ROWS_PER_TILE // ZR

    mesh = plsc.VectorSubcoreMesh(core_axis_name="c", subcore_axis_name="s")

    @functools.partial(
        pl.kernel,
        out_type=[
            jax.ShapeDtypeStruct((na_pad, HH), jnp.float32),
            jax.ShapeDtypeStruct((na_pad, HH), jnp.float32),
            jax.ShapeDtypeStruct((NS * na_pad,), jnp.float32),
        ],
        mesh=mesh,
        compiler_params=pltpu.CompilerParams(needs_layout_passes=False),
        scratch_types=[
            pltpu.VMEM((IB,), jnp.int32),
            pltpu.VMEM((IB,), jnp.int32),
            pltpu.VMEM((G,), jnp.int32),
            pltpu.VMEM((G,), jnp.int32),
            pltpu.VMEM((G, HH), jnp.float32),
            pltpu.VMEM((G, HH), jnp.float32),
            pltpu.VMEM((na_pad,), jnp.float32),
            pltpu.VMEM_SHARED((na_pad, HH), jnp.float32),
            pltpu.SemaphoreType.DMA,
            pltpu.SemaphoreType.DMA,
        ],
    )
    def kern(ph0_hbm, ph1_hbm, ea_hbm, ep_hbm, zrow_hbm, zhist_hbm,
             s0_hbm, s1_hbm, cnt_hbm,
             ablk_v, pblk_v, aidx0_v, aidx1_v, rows0_v, rows1_v,
             hist_v, acc_sh, sem0, sem1):
        cid = lax.axis_index("c")
        sid = lax.axis_index("s")
        one16 = jnp.ones((16,), jnp.float32)
        aidx = (aidx0_v, aidx1_v)
        rows = (rows0_v, rows1_v)
        sems = (sem0, sem1)

        pltpu.sync_copy(zrow_hbm, rows0_v)
        pltpu.sync_copy(zhist_hbm, hist_v)
        row_lo = sid * ROWS_PER_TILE
        for k in range(n_zc):
            pltpu.sync_copy(rows0_v, acc_sh.at[pl.ds(row_lo + k * ZR, ZR)])
        plsc.subcore_barrier()

        edge_lo = sid * E_PER

        def run_accum(ph_hbm, do_hist):
            def fetch(g, b):
                @pl.when(lax.rem(g, CPB) == 0)
                def _():
                    base = edge_lo + g * G
                    pltpu.sync_copy(ea_hbm.at[pl.ds(base, IB)], ablk_v)
                    pltpu.sync_copy(ep_hbm.at[pl.ds(base, IB)], pblk_v)
                off = lax.rem(g, CPB) * G
                for i in range(G // 16):
                    av = ablk_v[pl.ds(off + i * 16, 16)]
                    aidx[b][pl.ds(i * 16, 16)] = av
                    if do_hist:
                        plsc.addupdate_scatter(hist_v, [av], one16)
                pltpu.async_copy(ph_hbm.at[pblk_v.at[pl.ds(off, G)]],
                                 rows[b], sems[b])

            def process(g, b, prefetch):
                off = lax.rem(g, CPB) * G
                pltpu.make_async_copy(ph_hbm.at[pblk_v.at[pl.ds(off, G)]],
                                      rows[b], sems[b]).wait()
                if prefetch:
                    fetch(g + 1, 1 - b)
                pltpu.sync_copy(rows[b], acc_sh.at[aidx[b]], add=True)

            fetch(0, 0)

            def pair(k, _):
                g = 2 * k
                process(g, 0, True)
                process(g + 1, 1, True)
                return 0
            n_pairs = (n_chunks - 1) // 2
            lax.fori_loop(0, n_pairs, pair, 0)
            for g in range(2 * n_pairs, n_chunks):
                process(g, g % 2, g < n_chunks - 1)

        @pl.when(cid == 0)
        def _():
            run_accum(ph0_hbm, True)

        @pl.when(cid == 1)
        def _():
            run_accum(ph1_hbm, False)

        plsc.subcore_barrier()

        def copy_out(dst_hbm):
            for k in range(n_zc):
                sl = pl.ds(row_lo + k * ZR, ZR)
                pltpu.sync_copy(acc_sh.at[sl], rows0_v)
                pltpu.sync_copy(rows0_v, dst_hbm.at[sl])

        @pl.when(cid == 0)
        def _():
            copy_out(s0_hbm)
            pltpu.sync_copy(hist_v, cnt_hbm.at[pl.ds(sid * na_pad, na_pad)])

        @pl.when(cid == 1)
        def _():
            copy_out(s1_hbm)

    zrow = jnp.zeros((ZR, HH), jnp.float32)
    zhist = jnp.zeros((na_pad,), jnp.float32)
    pad = jnp.zeros((IB,), jnp.int32)
    ea_pad = jnp.concatenate([e_author, pad])
    ep_pad = jnp.concatenate([e_paper, pad])
    s0, s1, cnt_flat = kern(ph0, ph1, ea_pad, ep_pad, zrow, zhist)
    return s0, s1, cnt_flat.reshape(NS, na_pad)


def _author_linear(s0, s1, cnt, W_author, b_author2d):
    NA, HH = s0.shape
    H = W_author.shape[0]
    NT = cnt.shape[0]
    BLK = next(b for b in range(2048, 7, -8) if NA % b == 0)

    def body(s0_ref, s1_ref, c_ref, w_ref, b_ref, out_ref):
        w = w_ref[...]
        h = lax.dot_general(s0_ref[...], w[:, :HH], (((1,), (1,)), ((), ())),
                            preferred_element_type=jnp.float32)
        h = h + lax.dot_general(s1_ref[...], w[:, HH:], (((1,), (1,)), ((), ())),
                                preferred_element_type=jnp.float32)
        counts = jnp.sum(c_ref[...], axis=0)[:, None]
        scale = 1.0 / jnp.maximum(counts, 1.0)
        out_ref[...] = h * scale + b_ref[...]

    return pl.pallas_call(
        body,
        grid=(NA // BLK,),
        in_specs=[
            pl.BlockSpec((BLK, HH), lambda i: (i, 0)),
            pl.BlockSpec((BLK, HH), lambda i: (i, 0)),
            pl.BlockSpec((NT, BLK), lambda i: (0, i)),
            pl.BlockSpec((H, H), lambda i: (0, 0)),
            pl.BlockSpec((1, H), lambda i: (0, 0)),
        ],
        out_specs=pl.BlockSpec((BLK, H), lambda i: (i, 0)),
        out_shape=jax.ShapeDtypeStruct((NA, H), jnp.float32),
    )(s0, s1, cnt, W_author, b_author2d)


def _score_sc(author_h, paper_h, s_author, s_paper):
    NSUP = s_author.shape[0]
    H = author_h.shape[1]
    NC, NS = _sc_geometry()
    NW = NC * NS
    S_PER = NSUP // NW
    assert S_PER * NW == NSUP
    G = _chunk_size(S_PER)
    n_chunks = S_PER // G
    GP = ((G + 15) // 16) * 16
    n_groups = GP // 16

    mesh = plsc.VectorSubcoreMesh(core_axis_name="c", subcore_axis_name="s")

    @functools.partial(
        pl.kernel,
        out_type=jax.ShapeDtypeStruct((NSUP,), jnp.float32),
        mesh=mesh,
        compiler_params=pltpu.CompilerParams(needs_layout_passes=False),
        scratch_types=[
            pltpu.VMEM((S_PER,), jnp.int32),
            pltpu.VMEM((S_PER,), jnp.int32),
            pltpu.VMEM((G, H), jnp.float32),
            pltpu.VMEM((G, H), jnp.float32),
            pltpu.VMEM((G, H), jnp.float32),
            pltpu.VMEM((G, H), jnp.float32),
            pltpu.VMEM((GP * 16,), jnp.float32),
            pltpu.VMEM((S_PER + 16,), jnp.float32),
            pltpu.SemaphoreType.DMA,
            pltpu.SemaphoreType.DMA,
        ],
    )
    def kern(ah_hbm, ph_hbm, sa_hbm, sp_hbm, out_hbm,
             aidx_v, pidx_v,
             arows0_v, arows1_v, prows0_v, prows1_v,
             part_v, outb_v, sem0, sem1):
        cid = lax.axis_index("c")
        sid = lax.axis_index("s")
        wid = sid * NC + cid
        base0 = wid * S_PER
        lane = lax.iota(jnp.int32, 16)
        arows = (arows0_v, arows1_v)
        prows = (prows0_v, prows1_v)
        sems = (sem0, sem1)

        pltpu.sync_copy(sa_hbm.at[pl.ds(base0, S_PER)], aidx_v)
        pltpu.sync_copy(sp_hbm.at[pl.ds(base0, S_PER)], pidx_v)

        def fetch(g, b):
            off = g * G
            pltpu.async_copy(ah_hbm.at[aidx_v.at[pl.ds(off, G)]],
                             arows[b], sems[b])
            pltpu.async_copy(ph_hbm.at[pidx_v.at[pl.ds(off, G)]],
                             prows[b], sems[b])

        def process(g, b, prefetch):
            off = g * G
            pltpu.make_async_copy(ah_hbm.at[aidx_v.at[pl.ds(off, G)]],
                                  arows[b], sems[b]).wait()
            pltpu.make_async_copy(ph_hbm.at[pidx_v.at[pl.ds(off, G)]],
                                  prows[b], sems[b]).wait()
            if prefetch:
                fetch(g + 1, 1 - b)

            def edge(e, _):
                acc = jnp.zeros((16,), jnp.float32)
                for c in range(H // 16):
                    sl = pl.ds(c * 16, 16)
                    acc = acc + arows[b][e, sl] * prows[b][e, sl]
                part_v[pl.ds(e * 16, 16)] = acc
                return 0
            lax.fori_loop(0, G, edge, 0)

            for grp in range(n_groups):
                rows = (jnp.full((16,), grp * 16, jnp.int32) + lane) * 16
                tot = jnp.zeros((16,), jnp.float32)
                for j in range(16):
                    tot = tot + plsc.load_gather(part_v, [rows + j])
                outb_v[pl.ds(off + grp * 16, 16)] = tot

        fetch(0, 0)

        def pair(k, _):
            g = 2 * k
            process(g, 0, True)
            process(g + 1, 1, True)
            return 0
        n_pairs = (n_chunks - 1) // 2
        lax.fori_loop(0, n_pairs, pair, 0)
        for g in range(2 * n_pairs, n_chunks):
            process(g, g % 2, g < n_chunks - 1)

        pltpu.sync_copy(outb_v.at[pl.ds(0, S_PER)],
                        out_hbm.at[pl.ds(base0, S_PER)])

    return kern(author_h, paper_h, s_author, s_paper)


def kernel(x_author, x_paper, edge_index, supervision_edge_index,
           W_paper, b_paper, W_author, b_author):
    n_authors = x_author.shape[0]
    ei = edge_index.astype(jnp.int32)
    si = supervision_edge_index.astype(jnp.int32)
    e_author, e_paper = ei[0], ei[1]
    s_author, s_paper = si[0], si[1]

    paper_h, ph0, ph1 = _paper_linear(x_paper, W_paper,
                                      b_paper.reshape(1, -1))
    s0, s1, cnt = _segment_sum_sc(ph0, ph1, e_author, e_paper, n_authors)
    author_h = _author_linear(s0, s1, cnt, W_author, b_author.reshape(1, -1))
    return _score_sc(author_h, paper_h, s_author, s_paper)


# speedup vs baseline: 4.9232x; 1.0859x over previous
import functools

import jax
import jax.numpy as jnp
from jax import lax
from jax.experimental import pallas as pl
from jax.experimental.pallas import tpu as pltpu
from jax.experimental.pallas import tpu_sc as plsc


def _sc_geometry():
    try:
        info = plsc.get_sparse_core_info()
        return int(info.num_cores), int(info.num_subcores)
    except Exception:
        return 2, 16


def _chunk_size(total, limit=128):
    for g in range(limit, 7, -8):
        if total % g == 0:
            return g
    raise ValueError(f"no valid chunk size for {total}")


def _paper_linear(x_paper, W_paper, b_paper2d):
    NP, PIN = x_paper.shape
    H = W_paper.shape[0]
    HH = H // 2
    BLK = 2000

    def body(x_ref, w_ref, b_ref, full_ref, h0_ref, h1_ref):
        h = lax.dot_general(x_ref[...], w_ref[...], (((1,), (1,)), ((), ())),
                            preferred_element_type=jnp.float32)
        h = h + b_ref[...]
        full_ref[...] = h
        h0_ref[...] = h[:, :HH]
        h1_ref[...] = h[:, HH:]

    return pl.pallas_call(
        body,
        grid=(NP // BLK,),
        in_specs=[
            pl.BlockSpec((BLK, PIN), lambda i: (i, 0)),
            pl.BlockSpec((H, PIN), lambda i: (0, 0)),
            pl.BlockSpec((1, H), lambda i: (0, 0)),
        ],
        out_specs=[
            pl.BlockSpec((BLK, H), lambda i: (i, 0)),
            pl.BlockSpec((BLK, HH), lambda i: (i, 0)),
            pl.BlockSpec((BLK, HH), lambda i: (i, 0)),
        ],
        out_shape=[
            jax.ShapeDtypeStruct((NP, H), jnp.float32),
            jax.ShapeDtypeStruct((NP, HH), jnp.float32),
            jax.ShapeDtypeStruct((NP, HH), jnp.float32),
        ],
    )(x_paper, W_paper, b_paper2d)


def _author_pad(n_authors, NS, ZR):
    rpt = -(-n_authors // NS)
    rpt = -(-rpt // ZR) * ZR
    return rpt, rpt * NS


def _count_sc(e_author, n_authors):
    NE = e_author.shape[0]
    NC, NS = _sc_geometry()
    NW = NC * NS
    E_PER = NE // NW
    assert E_PER * NW == NE
    _, na_pad = _author_pad(n_authors, NS, _chunk_size(NE // NS))
    n_full = E_PER // 16
    rem = E_PER - n_full * 16
    EPAD = E_PER + (16 - rem) % 16

    mesh = plsc.VectorSubcoreMesh(core_axis_name="c", subcore_axis_name="s")

    @functools.partial(
        pl.kernel,
        out_type=jax.ShapeDtypeStruct((NW * na_pad,), jnp.float32),
        mesh=mesh,
        compiler_params=pltpu.CompilerParams(needs_layout_passes=False),
        scratch_types=[
            pltpu.VMEM((EPAD,), jnp.int32),
            pltpu.VMEM((na_pad,), jnp.float32),
            pltpu.SemaphoreType.DMA,
        ],
    )
    def kern(ea_hbm, zhist_hbm, cnt_hbm, ids_v, hist_v, sem):
        cid = lax.axis_index("c")
        sid = lax.axis_index("s")
        wid = sid * NC + cid
        one16 = jnp.ones((16,), jnp.float32)
        lane = lax.iota(jnp.int32, 16)

        pltpu.sync_copy(zhist_hbm, hist_v)
        pltpu.sync_copy(ea_hbm.at[pl.ds(wid * E_PER, E_PER)],
                        ids_v.at[pl.ds(0, E_PER)])

        def step(i, _):
            av = ids_v[pl.ds(i * 16, 16)]
            plsc.addupdate_scatter(hist_v, [av], one16)
            return 0
        lax.fori_loop(0, n_full, step, 0)
        if rem:
            av = ids_v[pl.ds(n_full * 16, 16)]
            plsc.addupdate_scatter(hist_v, [av], one16, mask=lane < rem)

        pltpu.sync_copy(hist_v, cnt_hbm.at[pl.ds(wid * na_pad, na_pad)])

    zhist = jnp.zeros((na_pad,), jnp.float32)
    return kern(e_author, zhist).reshape(NW, na_pad)


def _segment_sum_sc(ph0, ph1, e_author, e_paper, n_authors):
    NE = e_author.shape[0]
    HH = ph0.shape[1]
    NC, NS = _sc_geometry()
    E_PER = NE // NS
    assert E_PER * NS == NE
    G = _chunk_size(E_PER)
    n_chunks = E_PER // G
    CPB = 25
    IB = CPB * G
    ZR = G
    ROWS_PER_TILE, na_pad = _author_pad(n_authors, NS, ZR)
    n_zc = ROWS_PER_TILE // ZR

    mesh = plsc.VectorSubcoreMesh(core_axis_name="c", subcore_axis_name="s")

    @functools.partial(
        pl.kernel,
        out_type=[
            jax.ShapeDtypeStruct((na_pad, HH), jnp.float32),
            jax.ShapeDtypeStruct((na_pad, HH), jnp.float32),
        ],
        mesh=mesh,
        compiler_params=pltpu.CompilerParams(needs_layout_passes=False),
        scratch_types=[
            pltpu.VMEM((IB,), jnp.int32),
            pltpu.VMEM((IB,), jnp.int32),
            pltpu.VMEM((G,), jnp.int32),
            pltpu.VMEM((G,), jnp.int32),
            pltpu.VMEM((G, HH), jnp.float32),
            pltpu.VMEM((G, HH), jnp.float32),
            pltpu.VMEM_SHARED((na_pad, HH), jnp.float32),
            pltpu.SemaphoreType.DMA,
            pltpu.SemaphoreType.DMA,
        ],
    )
    def kern(ph0_hbm, ph1_hbm, ea_hbm, ep_hbm, zrow_hbm,
             s0_hbm, s1_hbm,
             ablk_v, pblk_v, aidx0_v, aidx1_v, rows0_v, rows1_v,
             acc_sh, sem0, sem1):
        cid = lax.axis_index("c")
        sid = lax.axis_index("s")
        aidx = (aidx0_v, aidx1_v)
        rows = (rows0_v, rows1_v)
        sems = (sem0, sem1)

        pltpu.sync_copy(zrow_hbm, rows0_v)
        row_lo = sid * ROWS_PER_TILE
        for k in range(n_zc):
            pltpu.sync_copy(rows0_v, acc_sh.at[pl.ds(row_lo + k * ZR, ZR)])
        plsc.subcore_barrier()

        edge_lo = sid * E_PER

        def run_accum(ph_hbm):
            def fetch(g, b):
                @pl.when(lax.rem(g, CPB) == 0)
                def _():
                    base = edge_lo + g * G
                    pltpu.sync_copy(ea_hbm.at[pl.ds(base, IB)], ablk_v)
                    pltpu.sync_copy(ep_hbm.at[pl.ds(base, IB)], pblk_v)
                off = lax.rem(g, CPB) * G
                for i in range(G // 16):
                    aidx[b][pl.ds(i * 16, 16)] = ablk_v[pl.ds(off + i * 16, 16)]
                pltpu.async_copy(ph_hbm.at[pblk_v.at[pl.ds(off, G)]],
                                 rows[b], sems[b])

            def process(g, b, prefetch):
                off = lax.rem(g, CPB) * G
                pltpu.make_async_copy(ph_hbm.at[pblk_v.at[pl.ds(off, G)]],
                                      rows[b], sems[b]).wait()
                if prefetch:
                    fetch(g + 1, 1 - b)
                pltpu.sync_copy(rows[b], acc_sh.at[aidx[b]], add=True)

            fetch(0, 0)

            def pair(k, _):
                g = 2 * k
                process(g, 0, True)
                process(g + 1, 1, True)
                return 0
            n_pairs = (n_chunks - 1) // 2
            lax.fori_loop(0, n_pairs, pair, 0)
            for g in range(2 * n_pairs, n_chunks):
                process(g, g % 2, g < n_chunks - 1)

        @pl.when(cid == 0)
        def _():
            run_accum(ph0_hbm)

        @pl.when(cid == 1)
        def _():
            run_accum(ph1_hbm)

        plsc.subcore_barrier()

        def copy_out(dst_hbm):
            for k in range(n_zc):
                sl = pl.ds(row_lo + k * ZR, ZR)
                pltpu.sync_copy(acc_sh.at[sl], rows0_v)
                pltpu.sync_copy(rows0_v, dst_hbm.at[sl])

        @pl.when(cid == 0)
        def _():
            copy_out(s0_hbm)

        @pl.when(cid == 1)
        def _():
            copy_out(s1_hbm)

    zrow = jnp.zeros((ZR, HH), jnp.float32)
    pad = jnp.zeros((IB,), jnp.int32)
    ea_pad = jnp.concatenate([e_author, pad])
    ep_pad = jnp.concatenate([e_paper, pad])
    return kern(ph0, ph1, ea_pad, ep_pad, zrow)


def _author_linear(s0, s1, cnt, W_author, b_author2d):
    NA, HH = s0.shape
    H = W_author.shape[0]
    NT = cnt.shape[0]
    BLK = next(b for b in range(2048, 7, -8) if NA % b == 0)

    def body(s0_ref, s1_ref, c_ref, w_ref, b_ref, out_ref):
        w = w_ref[...]
        h = lax.dot_general(s0_ref[...], w[:, :HH], (((1,), (1,)), ((), ())),
                            preferred_element_type=jnp.float32)
        h = h + lax.dot_general(s1_ref[...], w[:, HH:], (((1,), (1,)), ((), ())),
                                preferred_element_type=jnp.float32)
        counts = jnp.sum(c_ref[...], axis=0)[:, None]
        scale = 1.0 / jnp.maximum(counts, 1.0)
        out_ref[...] = h * scale + b_ref[...]

    return pl.pallas_call(
        body,
        grid=(NA // BLK,),
        in_specs=[
            pl.BlockSpec((BLK, HH), lambda i: (i, 0)),
            pl.BlockSpec((BLK, HH), lambda i: (i, 0)),
            pl.BlockSpec((NT, BLK), lambda i: (0, i)),
            pl.BlockSpec((H, H), lambda i: (0, 0)),
            pl.BlockSpec((1, H), lambda i: (0, 0)),
        ],
        out_specs=pl.BlockSpec((BLK, H), lambda i: (i, 0)),
        out_shape=jax.ShapeDtypeStruct((NA, H), jnp.float32),
    )(s0, s1, cnt, W_author, b_author2d)


def _score_sc(author_h, paper_h, s_author, s_paper):
    NSUP = s_author.shape[0]
    H = author_h.shape[1]
    NC, NS = _sc_geometry()
    NW = NC * NS
    S_PER = NSUP // NW
    assert S_PER * NW == NSUP
    G = _chunk_size(S_PER)
    n_chunks = S_PER // G
    GP = ((G + 15) // 16) * 16
    n_groups = GP // 16

    mesh = plsc.VectorSubcoreMesh(core_axis_name="c", subcore_axis_name="s")

    @functools.partial(
        pl.kernel,
        out_type=jax.ShapeDtypeStruct((NSUP,), jnp.float32),
        mesh=mesh,
        compiler_params=pltpu.CompilerParams(needs_layout_passes=False),
        scratch_types=[
            pltpu.VMEM((S_PER,), jnp.int32),
            pltpu.VMEM((S_PER,), jnp.int32),
            pltpu.VMEM((G, H), jnp.float32),
            pltpu.VMEM((G, H), jnp.float32),
            pltpu.VMEM((G, H), jnp.float32),
            pltpu.VMEM((G, H), jnp.float32),
            pltpu.VMEM((GP * 16,), jnp.float32),
            pltpu.VMEM((S_PER + 16,), jnp.float32),
            pltpu.SemaphoreType.DMA,
            pltpu.SemaphoreType.DMA,
        ],
    )
    def kern(ah_hbm, ph_hbm, sa_hbm, sp_hbm, out_hbm,
             aidx_v, pidx_v,
             arows0_v, arows1_v, prows0_v, prows1_v,
             part_v, outb_v, sem0, sem1):
        cid = lax.axis_index("c")
        sid = lax.axis_index("s")
        wid = sid * NC + cid
        base0 = wid * S_PER
        lane = lax.iota(jnp.int32, 16)
        arows = (arows0_v, arows1_v)
        prows = (prows0_v, prows1_v)
        sems = (sem0, sem1)

        pltpu.sync_copy(sa_hbm.at[pl.ds(base0, S_PER)], aidx_v)
        pltpu.sync_copy(sp_hbm.at[pl.ds(base0, S_PER)], pidx_v)

        def fetch(g, b):
            off = g * G
            pltpu.async_copy(ah_hbm.at[aidx_v.at[pl.ds(off, G)]],
                             arows[b], sems[b])
            pltpu.async_copy(ph_hbm.at[pidx_v.at[pl.ds(off, G)]],
                             prows[b], sems[b])

        def process(g, b, prefetch):
            off = g * G
            pltpu.make_async_copy(ah_hbm.at[aidx_v.at[pl.ds(off, G)]],
                                  arows[b], sems[b]).wait()
            pltpu.make_async_copy(ph_hbm.at[pidx_v.at[pl.ds(off, G)]],
                                  prows[b], sems[b]).wait()
            if prefetch:
                fetch(g + 1, 1 - b)

            @plsc.parallel_loop(0, G, unroll=2)
            def _(e):
                accs = [jnp.zeros((16,), jnp.float32) for _ in range(4)]
                for c in range(H // 16):
                    sl = pl.ds(c * 16, 16)
                    accs[c % 4] = accs[c % 4] + arows[b][e, sl] * prows[b][e, sl]
                part_v[pl.ds(e * 16, 16)] = (accs[0] + accs[1]) + (accs[2] + accs[3])

            for grp in range(n_groups):
                rows = (jnp.full((16,), grp * 16, jnp.int32) + lane) * 16
                tot = jnp.zeros((16,), jnp.float32)
                for j in range(16):
                    tot = tot + plsc.load_gather(part_v, [rows + j])
                outb_v[pl.ds(off + grp * 16, 16)] = tot

        fetch(0, 0)

        def pair(k, _):
            g = 2 * k
            process(g, 0, True)
            process(g + 1, 1, True)
            return 0
        n_pairs = (n_chunks - 1) // 2
        lax.fori_loop(0, n_pairs, pair, 0)
        for g in range(2 * n_pairs, n_chunks):
            process(g, g % 2, g < n_chunks - 1)

        pltpu.sync_copy(outb_v.at[pl.ds(0, S_PER)],
                        out_hbm.at[pl.ds(base0, S_PER)])

    return kern(author_h, paper_h, s_author, s_paper)


def kernel(x_author, x_paper, edge_index, supervision_edge_index,
           W_paper, b_paper, W_author, b_author):
    n_authors = x_author.shape[0]
    ei = edge_index.astype(jnp.int32)
    si = supervision_edge_index.astype(jnp.int32)
    e_author, e_paper = ei[0], ei[1]
    s_author, s_paper = si[0], si[1]

    paper_h, ph0, ph1 = _paper_linear(x_paper, W_paper,
                                      b_paper.reshape(1, -1))
    cnt = _count_sc(e_author, n_authors)
    s0, s1 = _segment_sum_sc(ph0, ph1, e_author, e_paper, n_authors)
    author_h = _author_linear(s0, s1, cnt, W_author, b_author.reshape(1, -1))
    return _score_sc(author_h, paper_h, s_author, s_paper)


# speedup vs baseline: 5.7563x; 1.1692x over previous
import functools

import jax
import jax.numpy as jnp
from jax import lax
from jax.experimental import pallas as pl
from jax.experimental.pallas import tpu as pltpu
from jax.experimental.pallas import tpu_sc as plsc


def _sc_geometry():
    try:
        info = plsc.get_sparse_core_info()
        return int(info.num_cores), int(info.num_subcores)
    except Exception:
        return 2, 16


def _chunk_size(total, limit=128):
    for g in range(limit, 7, -8):
        if total % g == 0:
            return g
    raise ValueError(f"no valid chunk size for {total}")


def _pack_bf16_pairs(lo, hi):
    lo16 = lax.bitcast_convert_type(lo.astype(jnp.bfloat16), jnp.int16)
    hi16 = lax.bitcast_convert_type(hi.astype(jnp.bfloat16), jnp.int16)
    lo32 = lo16.astype(jnp.int32) & jnp.int32(65535)
    hi32 = hi16.astype(jnp.int32) << 16
    return hi32 | lo32


def _paper_linear(x_paper, W_paper, b_paper2d):
    NP, PIN = x_paper.shape
    H = W_paper.shape[0]
    HH = H // 2
    BLK = 2000

    def body(x_ref, w_ref, b_ref, full_ref, h0_ref, h1_ref):
        h = lax.dot_general(x_ref[...], w_ref[...], (((1,), (1,)), ((), ())),
                            preferred_element_type=jnp.float32)
        h = h + b_ref[...]
        full_ref[...] = _pack_bf16_pairs(h[:, :HH], h[:, HH:])
        h0_ref[...] = h[:, :HH]
        h1_ref[...] = h[:, HH:]

    return pl.pallas_call(
        body,
        grid=(NP // BLK,),
        in_specs=[
            pl.BlockSpec((BLK, PIN), lambda i: (i, 0)),
            pl.BlockSpec((H, PIN), lambda i: (0, 0)),
            pl.BlockSpec((1, H), lambda i: (0, 0)),
        ],
        out_specs=[
            pl.BlockSpec((BLK, HH), lambda i: (i, 0)),
            pl.BlockSpec((BLK, HH), lambda i: (i, 0)),
            pl.BlockSpec((BLK, HH), lambda i: (i, 0)),
        ],
        out_shape=[
            jax.ShapeDtypeStruct((NP, HH), jnp.int32),
            jax.ShapeDtypeStruct((NP, HH), jnp.float32),
            jax.ShapeDtypeStruct((NP, HH), jnp.float32),
        ],
    )(x_paper, W_paper, b_paper2d)


def _author_pad(n_authors, NS, ZR):
    rpt = -(-n_authors // NS)
    rpt = -(-rpt // ZR) * ZR
    return rpt, rpt * NS


def _count_sc(e_author, n_authors):
    NE = e_author.shape[0]
    NC, NS = _sc_geometry()
    NW = NC * NS
    E_PER = NE // NW
    assert E_PER * NW == NE
    _, na_pad = _author_pad(n_authors, NS, _chunk_size(NE // NS))
    n_full = E_PER // 16
    rem = E_PER - n_full * 16
    EPAD = E_PER + (16 - rem) % 16

    mesh = plsc.VectorSubcoreMesh(core_axis_name="c", subcore_axis_name="s")

    @functools.partial(
        pl.kernel,
        out_type=jax.ShapeDtypeStruct((NW * na_pad,), jnp.float32),
        mesh=mesh,
        compiler_params=pltpu.CompilerParams(needs_layout_passes=False),
        scratch_types=[
            pltpu.VMEM((EPAD,), jnp.int32),
            pltpu.VMEM((na_pad,), jnp.float32),
            pltpu.SemaphoreType.DMA,
        ],
    )
    def kern(ea_hbm, zhist_hbm, cnt_hbm, ids_v, hist_v, sem):
        cid = lax.axis_index("c")
        sid = lax.axis_index("s")
        wid = sid * NC + cid
        one16 = jnp.ones((16,), jnp.float32)
        lane = lax.iota(jnp.int32, 16)

        pltpu.sync_copy(zhist_hbm, hist_v)
        pltpu.sync_copy(ea_hbm.at[pl.ds(wid * E_PER, E_PER)],
                        ids_v.at[pl.ds(0, E_PER)])

        def step(i, _):
            av = ids_v[pl.ds(i * 16, 16)]
            plsc.addupdate_scatter(hist_v, [av], one16)
            return 0
        lax.fori_loop(0, n_full, step, 0)
        if rem:
            av = ids_v[pl.ds(n_full * 16, 16)]
            plsc.addupdate_scatter(hist_v, [av], one16, mask=lane < rem)

        pltpu.sync_copy(hist_v, cnt_hbm.at[pl.ds(wid * na_pad, na_pad)])

    zhist = jnp.zeros((na_pad,), jnp.float32)
    return kern(e_author, zhist).reshape(NW, na_pad)


def _segment_sum_sc(ph0, ph1, e_author, e_paper, n_authors):
    NE = e_author.shape[0]
    HH = ph0.shape[1]
    NC, NS = _sc_geometry()
    E_PER = NE // NS
    assert E_PER * NS == NE
    G = _chunk_size(E_PER)
    n_chunks = E_PER // G
    CPB = 25
    IB = CPB * G
    ZR = G
    ROWS_PER_TILE, na_pad = _author_pad(n_authors, NS, ZR)
    n_zc = ROWS_PER_TILE // ZR

    mesh = plsc.VectorSubcoreMesh(core_axis_name="c", subcore_axis_name="s")

    @functools.partial(
        pl.kernel,
        out_type=[
            jax.ShapeDtypeStruct((na_pad, HH), jnp.float32),
            jax.ShapeDtypeStruct((na_pad, HH), jnp.float32),
        ],
        mesh=mesh,
        compiler_params=pltpu.CompilerParams(needs_layout_passes=False),
        scratch_types=[
            pltpu.VMEM((IB,), jnp.int32),
            pltpu.VMEM((IB,), jnp.int32),
            pltpu.VMEM((G,), jnp.int32),
            pltpu.VMEM((G,), jnp.int32),
            pltpu.VMEM((G, HH), jnp.float32),
            pltpu.VMEM((G, HH), jnp.float32),
            pltpu.VMEM_SHARED((na_pad, HH), jnp.float32),
            pltpu.SemaphoreType.DMA,
            pltpu.SemaphoreType.DMA,
        ],
    )
    def kern(ph0_hbm, ph1_hbm, ea_hbm, ep_hbm, zrow_hbm,
             s0_hbm, s1_hbm,
             ablk_v, pblk_v, aidx0_v, aidx1_v, rows0_v, rows1_v,
             acc_sh, sem0, sem1):
        cid = lax.axis_index("c")
        sid = lax.axis_index("s")
        aidx = (aidx0_v, aidx1_v)
        rows = (rows0_v, rows1_v)
        sems = (sem0, sem1)

        pltpu.sync_copy(zrow_hbm, rows0_v)
        row_lo = sid * ROWS_PER_TILE
        for k in range(n_zc):
            pltpu.sync_copy(rows0_v, acc_sh.at[pl.ds(row_lo + k * ZR, ZR)])
        plsc.subcore_barrier()

        edge_lo = sid * E_PER

        def run_accum(ph_hbm):
            def fetch(g, b):
                @pl.when(lax.rem(g, CPB) == 0)
                def _():
                    base = edge_lo + g * G
                    pltpu.sync_copy(ea_hbm.at[pl.ds(base, IB)], ablk_v)
                    pltpu.sync_copy(ep_hbm.at[pl.ds(base, IB)], pblk_v)
                off = lax.rem(g, CPB) * G
                for i in range(G // 16):
                    aidx[b][pl.ds(i * 16, 16)] = ablk_v[pl.ds(off + i * 16, 16)]
                pltpu.async_copy(ph_hbm.at[pblk_v.at[pl.ds(off, G)]],
                                 rows[b], sems[b])

            def process(g, b, prefetch):
                off = lax.rem(g, CPB) * G
                pltpu.make_async_copy(ph_hbm.at[pblk_v.at[pl.ds(off, G)]],
                                      rows[b], sems[b]).wait()
                if prefetch:
                    fetch(g + 1, 1 - b)
                pltpu.sync_copy(rows[b], acc_sh.at[aidx[b]], add=True)

            fetch(0, 0)

            def pair(k, _):
                g = 2 * k
                process(g, 0, True)
                process(g + 1, 1, True)
                return 0
            n_pairs = (n_chunks - 1) // 2
            lax.fori_loop(0, n_pairs, pair, 0)
            for g in range(2 * n_pairs, n_chunks):
                process(g, g % 2, g < n_chunks - 1)

        @pl.when(cid == 0)
        def _():
            run_accum(ph0_hbm)

        @pl.when(cid == 1)
        def _():
            run_accum(ph1_hbm)

        plsc.subcore_barrier()

        def copy_out(dst_hbm):
            for k in range(n_zc):
                sl = pl.ds(row_lo + k * ZR, ZR)
                pltpu.sync_copy(acc_sh.at[sl], rows0_v)
                pltpu.sync_copy(rows0_v, dst_hbm.at[sl])

        @pl.when(cid == 0)
        def _():
            copy_out(s0_hbm)

        @pl.when(cid == 1)
        def _():
            copy_out(s1_hbm)

    zrow = jnp.zeros((ZR, HH), jnp.float32)
    pad = jnp.zeros((IB,), jnp.int32)
    ea_pad = jnp.concatenate([e_author, pad])
    ep_pad = jnp.concatenate([e_paper, pad])
    return kern(ph0, ph1, ea_pad, ep_pad, zrow)


def _author_linear(s0, s1, cnt, W_author, b_author2d):
    NA, HH = s0.shape
    H = W_author.shape[0]
    NT = cnt.shape[0]
    BLK = next(b for b in range(2048, 7, -8) if NA % b == 0)

    def body(s0_ref, s1_ref, c_ref, w_ref, b_ref, out_ref):
        w = w_ref[...]
        h = lax.dot_general(s0_ref[...], w[:, :HH], (((1,), (1,)), ((), ())),
                            preferred_element_type=jnp.float32)
        h = h + lax.dot_general(s1_ref[...], w[:, HH:], (((1,), (1,)), ((), ())),
                                preferred_element_type=jnp.float32)
        counts = jnp.sum(c_ref[...], axis=0)[:, None]
        scale = 1.0 / jnp.maximum(counts, 1.0)
        au = h * scale + b_ref[...]
        out_ref[...] = _pack_bf16_pairs(au[:, :HH], au[:, HH:])

    return pl.pallas_call(
        body,
        grid=(NA // BLK,),
        in_specs=[
            pl.BlockSpec((BLK, HH), lambda i: (i, 0)),
            pl.BlockSpec((BLK, HH), lambda i: (i, 0)),
            pl.BlockSpec((NT, BLK), lambda i: (0, i)),
            pl.BlockSpec((H, H), lambda i: (0, 0)),
            pl.BlockSpec((1, H), lambda i: (0, 0)),
        ],
        out_specs=pl.BlockSpec((BLK, HH), lambda i: (i, 0)),
        out_shape=jax.ShapeDtypeStruct((NA, HH), jnp.int32),
    )(s0, s1, cnt, W_author, b_author2d)


def _score_sc(author_h, paper_h, s_author, s_paper):
    NSUP = s_author.shape[0]
    HP = author_h.shape[1]
    NC, NS = _sc_geometry()
    NW = NC * NS
    S_PER = NSUP // NW
    assert S_PER * NW == NSUP
    G = _chunk_size(S_PER)
    n_chunks = S_PER // G
    GP = ((G + 15) // 16) * 16
    n_groups = GP // 16

    mesh = plsc.VectorSubcoreMesh(core_axis_name="c", subcore_axis_name="s")

    @functools.partial(
        pl.kernel,
        out_type=jax.ShapeDtypeStruct((NSUP,), jnp.float32),
        mesh=mesh,
        compiler_params=pltpu.CompilerParams(needs_layout_passes=False),
        scratch_types=[
            pltpu.VMEM((S_PER,), jnp.int32),
            pltpu.VMEM((S_PER,), jnp.int32),
            pltpu.VMEM((G, HP), jnp.int32),
            pltpu.VMEM((G, HP), jnp.int32),
            pltpu.VMEM((G, HP), jnp.int32),
            pltpu.VMEM((G, HP), jnp.int32),
            pltpu.VMEM((GP * 16,), jnp.float32),
            pltpu.VMEM((S_PER + 16,), jnp.float32),
            pltpu.SemaphoreType.DMA,
            pltpu.SemaphoreType.DMA,
        ],
    )
    def kern(ah_hbm, ph_hbm, sa_hbm, sp_hbm, out_hbm,
             aidx_v, pidx_v,
             arows0_v, arows1_v, prows0_v, prows1_v,
             part_v, outb_v, sem0, sem1):
        cid = lax.axis_index("c")
        sid = lax.axis_index("s")
        wid = sid * NC + cid
        base0 = wid * S_PER
        lane = lax.iota(jnp.int32, 16)
        arows = (arows0_v, arows1_v)
        prows = (prows0_v, prows1_v)
        sems = (sem0, sem1)

        pltpu.sync_copy(sa_hbm.at[pl.ds(base0, S_PER)], aidx_v)
        pltpu.sync_copy(sp_hbm.at[pl.ds(base0, S_PER)], pidx_v)

        def fetch(g, b):
            off = g * G
            pltpu.async_copy(ah_hbm.at[aidx_v.at[pl.ds(off, G)]],
                             arows[b], sems[b])
            pltpu.async_copy(ph_hbm.at[pidx_v.at[pl.ds(off, G)]],
                             prows[b], sems[b])

        def process(g, b, prefetch):
            off = g * G
            pltpu.make_async_copy(ah_hbm.at[aidx_v.at[pl.ds(off, G)]],
                                  arows[b], sems[b]).wait()
            pltpu.make_async_copy(ph_hbm.at[pidx_v.at[pl.ds(off, G)]],
                                  prows[b], sems[b]).wait()
            if prefetch:
                fetch(g + 1, 1 - b)

            mask_hi = jnp.full((16,), -65536, jnp.int32)

            @plsc.parallel_loop(0, G, unroll=2)
            def _(e):
                accs = [jnp.zeros((16,), jnp.float32) for _ in range(4)]
                for c in range(HP // 16):
                    sl = pl.ds(c * 16, 16)
                    wa = arows[b][e, sl]
                    wp = prows[b][e, sl]
                    a_lo = plsc.bitcast(wa << 16, jnp.float32)
                    p_lo = plsc.bitcast(wp << 16, jnp.float32)
                    a_hi = plsc.bitcast(wa & mask_hi, jnp.float32)
                    p_hi = plsc.bitcast(wp & mask_hi, jnp.float32)
                    k = 2 * (c % 2)
                    accs[k] = accs[k] + a_lo * p_lo
                    accs[k + 1] = accs[k + 1] + a_hi * p_hi
                part_v[pl.ds(e * 16, 16)] = (accs[0] + accs[1]) + (accs[2] + accs[3])

            for grp in range(n_groups):
                rows = (jnp.full((16,), grp * 16, jnp.int32) + lane) * 16
                tot = jnp.zeros((16,), jnp.float32)
                for j in range(16):
                    tot = tot + plsc.load_gather(part_v, [rows + j])
                outb_v[pl.ds(off + grp * 16, 16)] = tot

        fetch(0, 0)

        def pair(k, _):
            g = 2 * k
            process(g, 0, True)
            process(g + 1, 1, True)
            return 0
        n_pairs = (n_chunks - 1) // 2
        lax.fori_loop(0, n_pairs, pair, 0)
        for g in range(2 * n_pairs, n_chunks):
            process(g, g % 2, g < n_chunks - 1)

        pltpu.sync_copy(outb_v.at[pl.ds(0, S_PER)],
                        out_hbm.at[pl.ds(base0, S_PER)])

    return kern(author_h, paper_h, s_author, s_paper)


def kernel(x_author, x_paper, edge_index, supervision_edge_index,
           W_paper, b_paper, W_author, b_author):
    n_authors = x_author.shape[0]
    ei = edge_index.astype(jnp.int32)
    si = supervision_edge_index.astype(jnp.int32)
    e_author, e_paper = ei[0], ei[1]
    s_author, s_paper = si[0], si[1]

    paper_h, ph0, ph1 = _paper_linear(x_paper, W_paper,
                                      b_paper.reshape(1, -1))
    cnt = _count_sc(e_author, n_authors)
    s0, s1 = _segment_sum_sc(ph0, ph1, e_author, e_paper, n_authors)
    author_h = _author_linear(s0, s1, cnt, W_author, b_author.reshape(1, -1))
    return _score_sc(author_h, paper_h, s_author, s_paper)


# speedup vs baseline: 5.7566x; 1.0001x over previous
import functools

import jax
import jax.numpy as jnp
from jax import lax
from jax.experimental import pallas as pl
from jax.experimental.pallas import tpu as pltpu
from jax.experimental.pallas import tpu_sc as plsc


def _sc_geometry():
    try:
        info = plsc.get_sparse_core_info()
        return int(info.num_cores), int(info.num_subcores)
    except Exception:
        return 2, 16


def _chunk_size(total, limit=128):
    for g in range(limit, 7, -8):
        if total % g == 0:
            return g
    raise ValueError(f"no valid chunk size for {total}")


def _pack_bf16_pairs(lo, hi):
    lo16 = lax.bitcast_convert_type(lo.astype(jnp.bfloat16), jnp.int16)
    hi16 = lax.bitcast_convert_type(hi.astype(jnp.bfloat16), jnp.int16)
    lo32 = lo16.astype(jnp.int32) & jnp.int32(65535)
    hi32 = hi16.astype(jnp.int32) << 16
    return hi32 | lo32


def _paper_linear(x_paper, W_paper, b_paper2d):
    NP, PIN = x_paper.shape
    H = W_paper.shape[0]
    HH = H // 2
    BLK = 2000

    def body(x_ref, w_ref, b_ref, full_ref, h0_ref, h1_ref):
        h = lax.dot_general(x_ref[...], w_ref[...], (((1,), (1,)), ((), ())),
                            preferred_element_type=jnp.float32)
        h = h + b_ref[...]
        full_ref[...] = _pack_bf16_pairs(h[:, :HH], h[:, HH:])
        h0_ref[...] = h[:, :HH]
        h1_ref[...] = h[:, HH:]

    return pl.pallas_call(
        body,
        grid=(NP // BLK,),
        in_specs=[
            pl.BlockSpec((BLK, PIN), lambda i: (i, 0)),
            pl.BlockSpec((H, PIN), lambda i: (0, 0)),
            pl.BlockSpec((1, H), lambda i: (0, 0)),
        ],
        out_specs=[
            pl.BlockSpec((BLK, HH), lambda i: (i, 0)),
            pl.BlockSpec((BLK, HH), lambda i: (i, 0)),
            pl.BlockSpec((BLK, HH), lambda i: (i, 0)),
        ],
        out_shape=[
            jax.ShapeDtypeStruct((NP, HH), jnp.int32),
            jax.ShapeDtypeStruct((NP, HH), jnp.float32),
            jax.ShapeDtypeStruct((NP, HH), jnp.float32),
        ],
    )(x_paper, W_paper, b_paper2d)


def _author_pad(n_authors, NS, ZR):
    rpt = -(-n_authors // NS)
    rpt = -(-rpt // ZR) * ZR
    return rpt, rpt * NS


def _count_sc(e_author, n_authors):
    NE = e_author.shape[0]
    NC, NS = _sc_geometry()
    NW = NC * NS
    E_PER = NE // NW
    assert E_PER * NW == NE
    _, na_pad = _author_pad(n_authors, NS, _chunk_size(NE // NS))
    n_full = E_PER // 16
    rem = E_PER - n_full * 16
    EPAD = E_PER + (16 - rem) % 16

    mesh = plsc.VectorSubcoreMesh(core_axis_name="c", subcore_axis_name="s")

    @functools.partial(
        pl.kernel,
        out_type=jax.ShapeDtypeStruct((NW * na_pad,), jnp.float32),
        mesh=mesh,
        compiler_params=pltpu.CompilerParams(needs_layout_passes=False),
        scratch_types=[
            pltpu.VMEM((EPAD,), jnp.int32),
            pltpu.VMEM((na_pad,), jnp.float32),
            pltpu.SemaphoreType.DMA,
        ],
    )
    def kern(ea_hbm, zhist_hbm, cnt_hbm, ids_v, hist_v, sem):
        cid = lax.axis_index("c")
        sid = lax.axis_index("s")
        wid = sid * NC + cid
        one16 = jnp.ones((16,), jnp.float32)
        lane = lax.iota(jnp.int32, 16)

        pltpu.sync_copy(zhist_hbm, hist_v)
        pltpu.sync_copy(ea_hbm.at[pl.ds(wid * E_PER, E_PER)],
                        ids_v.at[pl.ds(0, E_PER)])

        def step(i, _):
            av = ids_v[pl.ds(i * 16, 16)]
            plsc.addupdate_scatter(hist_v, [av], one16)
            return 0
        lax.fori_loop(0, n_full, step, 0)
        if rem:
            av = ids_v[pl.ds(n_full * 16, 16)]
            plsc.addupdate_scatter(hist_v, [av], one16, mask=lane < rem)

        pltpu.sync_copy(hist_v, cnt_hbm.at[pl.ds(wid * na_pad, na_pad)])

    zhist = jnp.zeros((na_pad,), jnp.float32)
    return kern(e_author, zhist).reshape(NW, na_pad)


def _segment_sum_sc(ph0, ph1, e_author, e_paper, n_authors):
    NE = e_author.shape[0]
    HH = ph0.shape[1]
    NC, NS = _sc_geometry()
    E_PER = NE // NS
    assert E_PER * NS == NE
    G = _chunk_size(E_PER)
    n_chunks = E_PER // G
    CPB = 25
    IB = CPB * G
    ZR = G
    ROWS_PER_TILE, na_pad = _author_pad(n_authors, NS, ZR)
    n_zc = ROWS_PER_TILE // ZR

    mesh = plsc.VectorSubcoreMesh(core_axis_name="c", subcore_axis_name="s")

    @functools.partial(
        pl.kernel,
        out_type=[
            jax.ShapeDtypeStruct((na_pad, HH), jnp.float32),
            jax.ShapeDtypeStruct((na_pad, HH), jnp.float32),
        ],
        mesh=mesh,
        compiler_params=pltpu.CompilerParams(needs_layout_passes=False),
        scratch_types=[
            pltpu.VMEM((IB,), jnp.int32),
            pltpu.VMEM((IB,), jnp.int32),
            pltpu.VMEM((G,), jnp.int32),
            pltpu.VMEM((G,), jnp.int32),
            pltpu.VMEM((G, HH), jnp.float32),
            pltpu.VMEM((G, HH), jnp.float32),
            pltpu.VMEM_SHARED((na_pad, HH), jnp.float32),
            pltpu.SemaphoreType.DMA,
            pltpu.SemaphoreType.DMA,
            pltpu.SemaphoreType.DMA,
            pltpu.SemaphoreType.DMA,
        ],
    )
    def kern(ph0_hbm, ph1_hbm, ea_hbm, ep_hbm, zrow_hbm,
             s0_hbm, s1_hbm,
             ablk_v, pblk_v, aidx0_v, aidx1_v, rows0_v, rows1_v,
             acc_sh, sem0, sem1, sem_s0, sem_s1):
        cid = lax.axis_index("c")
        sid = lax.axis_index("s")
        aidx = (aidx0_v, aidx1_v)
        rows = (rows0_v, rows1_v)
        sems = (sem0, sem1)
        ssems = (sem_s0, sem_s1)

        pltpu.sync_copy(zrow_hbm, rows0_v)
        row_lo = sid * ROWS_PER_TILE
        for k in range(n_zc):
            pltpu.sync_copy(rows0_v, acc_sh.at[pl.ds(row_lo + k * ZR, ZR)])
        plsc.subcore_barrier()

        edge_lo = sid * E_PER

        def run_accum(ph_hbm):
            def fetch(g, b):
                @pl.when(lax.rem(g, CPB) == 0)
                def _():
                    base = edge_lo + g * G
                    pltpu.sync_copy(ea_hbm.at[pl.ds(base, IB)], ablk_v)
                    pltpu.sync_copy(ep_hbm.at[pl.ds(base, IB)], pblk_v)

                @pl.when(g >= 2)
                def _():
                    pltpu.make_async_copy(rows[b], acc_sh.at[aidx[b]],
                                          ssems[b]).wait()
                off = lax.rem(g, CPB) * G
                for i in range(G // 16):
                    aidx[b][pl.ds(i * 16, 16)] = ablk_v[pl.ds(off + i * 16, 16)]
                pltpu.async_copy(ph_hbm.at[pblk_v.at[pl.ds(off, G)]],
                                 rows[b], sems[b])

            def process(g, b, prefetch):
                off = lax.rem(g, CPB) * G
                pltpu.make_async_copy(ph_hbm.at[pblk_v.at[pl.ds(off, G)]],
                                      rows[b], sems[b]).wait()
                if prefetch:
                    fetch(g + 1, 1 - b)
                pltpu.async_copy(rows[b], acc_sh.at[aidx[b]], ssems[b],
                                 add=True)

            fetch(0, 0)

            def pair(k, _):
                g = 2 * k
                process(g, 0, True)
                process(g + 1, 1, True)
                return 0
            assert n_chunks % 2 == 1 and n_chunks >= 3
            n_pairs = (n_chunks - 1) // 2
            lax.fori_loop(0, n_pairs, pair, 0)
            for g in range(2 * n_pairs, n_chunks):
                process(g, g % 2, g < n_chunks - 1)
            for b in range(2):
                pltpu.make_async_copy(rows[b], acc_sh.at[aidx[b]],
                                      ssems[b]).wait()

        @pl.when(cid == 0)
        def _():
            run_accum(ph0_hbm)

        @pl.when(cid == 1)
        def _():
            run_accum(ph1_hbm)

        plsc.subcore_barrier()

        def copy_out(dst_hbm):
            for k in range(n_zc):
                sl = pl.ds(row_lo + k * ZR, ZR)
                pltpu.sync_copy(acc_sh.at[sl], rows0_v)
                pltpu.sync_copy(rows0_v, dst_hbm.at[sl])

        @pl.when(cid == 0)
        def _():
            copy_out(s0_hbm)

        @pl.when(cid == 1)
        def _():
            copy_out(s1_hbm)

    zrow = jnp.zeros((ZR, HH), jnp.float32)
    pad = jnp.zeros((IB,), jnp.int32)
    ea_pad = jnp.concatenate([e_author, pad])
    ep_pad = jnp.concatenate([e_paper, pad])
    return kern(ph0, ph1, ea_pad, ep_pad, zrow)


def _author_linear(s0, s1, cnt, W_author, b_author2d):
    NA, HH = s0.shape
    H = W_author.shape[0]
    NT = cnt.shape[0]
    BLK = next(b for b in range(2048, 7, -8) if NA % b == 0)

    def body(s0_ref, s1_ref, c_ref, w_ref, b_ref, out_ref):
        w = w_ref[...]
        h = lax.dot_general(s0_ref[...], w[:, :HH], (((1,), (1,)), ((), ())),
                            preferred_element_type=jnp.float32)
        h = h + lax.dot_general(s1_ref[...], w[:, HH:], (((1,), (1,)), ((), ())),
                                preferred_element_type=jnp.float32)
        counts = jnp.sum(c_ref[...], axis=0)[:, None]
        scale = 1.0 / jnp.maximum(counts, 1.0)
        au = h * scale + b_ref[...]
        out_ref[...] = _pack_bf16_pairs(au[:, :HH], au[:, HH:])

    return pl.pallas_call(
        body,
        grid=(NA // BLK,),
        in_specs=[
            pl.BlockSpec((BLK, HH), lambda i: (i, 0)),
            pl.BlockSpec((BLK, HH), lambda i: (i, 0)),
            pl.BlockSpec((NT, BLK), lambda i: (0, i)),
            pl.BlockSpec((H, H), lambda i: (0, 0)),
            pl.BlockSpec((1, H), lambda i: (0, 0)),
        ],
        out_specs=pl.BlockSpec((BLK, HH), lambda i: (i, 0)),
        out_shape=jax.ShapeDtypeStruct((NA, HH), jnp.int32),
    )(s0, s1, cnt, W_author, b_author2d)


def _score_sc(author_h, paper_h, s_author, s_paper):
    NSUP = s_author.shape[0]
    HP = author_h.shape[1]
    NC, NS = _sc_geometry()
    NW = NC * NS
    S_PER = NSUP // NW
    assert S_PER * NW == NSUP
    G = _chunk_size(S_PER)
    n_chunks = S_PER // G
    GP = ((G + 15) // 16) * 16
    n_groups = GP // 16

    mesh = plsc.VectorSubcoreMesh(core_axis_name="c", subcore_axis_name="s")

    @functools.partial(
        pl.kernel,
        out_type=jax.ShapeDtypeStruct((NSUP,), jnp.float32),
        mesh=mesh,
        compiler_params=pltpu.CompilerParams(needs_layout_passes=False),
        scratch_types=[
            pltpu.VMEM((S_PER,), jnp.int32),
            pltpu.VMEM((S_PER,), jnp.int32),
            pltpu.VMEM((G, HP), jnp.int32),
            pltpu.VMEM((G, HP), jnp.int32),
            pltpu.VMEM((G, HP), jnp.int32),
            pltpu.VMEM((G, HP), jnp.int32),
            pltpu.VMEM((GP * 16,), jnp.float32),
            pltpu.VMEM((S_PER + 16,), jnp.float32),
            pltpu.SemaphoreType.DMA,
            pltpu.SemaphoreType.DMA,
        ],
    )
    def kern(ah_hbm, ph_hbm, sa_hbm, sp_hbm, out_hbm,
             aidx_v, pidx_v,
             arows0_v, arows1_v, prows0_v, prows1_v,
             part_v, outb_v, sem0, sem1):
        cid = lax.axis_index("c")
        sid = lax.axis_index("s")
        wid = sid * NC + cid
        base0 = wid * S_PER
        lane = lax.iota(jnp.int32, 16)
        arows = (arows0_v, arows1_v)
        prows = (prows0_v, prows1_v)
        sems = (sem0, sem1)

        pltpu.sync_copy(sa_hbm.at[pl.ds(base0, S_PER)], aidx_v)
        pltpu.sync_copy(sp_hbm.at[pl.ds(base0, S_PER)], pidx_v)

        def fetch(g, b):
            off = g * G
            pltpu.async_copy(ah_hbm.at[aidx_v.at[pl.ds(off, G)]],
                             arows[b], sems[b])
            pltpu.async_copy(ph_hbm.at[pidx_v.at[pl.ds(off, G)]],
                             prows[b], sems[b])

        def process(g, b, prefetch):
            off = g * G
            pltpu.make_async_copy(ah_hbm.at[aidx_v.at[pl.ds(off, G)]],
                                  arows[b], sems[b]).wait()
            pltpu.make_async_copy(ph_hbm.at[pidx_v.at[pl.ds(off, G)]],
                                  prows[b], sems[b]).wait()
            if prefetch:
                fetch(g + 1, 1 - b)

            mask_hi = jnp.full((16,), -65536, jnp.int32)

            @plsc.parallel_loop(0, G, unroll=2)
            def _(e):
                accs = [jnp.zeros((16,), jnp.float32) for _ in range(4)]
                for c in range(HP // 16):
                    sl = pl.ds(c * 16, 16)
                    wa = arows[b][e, sl]
                    wp = prows[b][e, sl]
                    a_lo = plsc.bitcast(wa << 16, jnp.float32)
                    p_lo = plsc.bitcast(wp << 16, jnp.float32)
                    a_hi = plsc.bitcast(wa & mask_hi, jnp.float32)
                    p_hi = plsc.bitcast(wp & mask_hi, jnp.float32)
                    k = 2 * (c % 2)
                    accs[k] = accs[k] + a_lo * p_lo
                    accs[k + 1] = accs[k + 1] + a_hi * p_hi
                part_v[pl.ds(e * 16, 16)] = (accs[0] + accs[1]) + (accs[2] + accs[3])

            for grp in range(n_groups):
                rows = (jnp.full((16,), grp * 16, jnp.int32) + lane) * 16
                tot = jnp.zeros((16,), jnp.float32)
                for j in range(16):
                    tot = tot + plsc.load_gather(part_v, [rows + j])
                outb_v[pl.ds(off + grp * 16, 16)] = tot

        fetch(0, 0)

        def pair(k, _):
            g = 2 * k
            process(g, 0, True)
            process(g + 1, 1, True)
            return 0
        n_pairs = (n_chunks - 1) // 2
        lax.fori_loop(0, n_pairs, pair, 0)
        for g in range(2 * n_pairs, n_chunks):
            process(g, g % 2, g < n_chunks - 1)

        pltpu.sync_copy(outb_v.at[pl.ds(0, S_PER)],
                        out_hbm.at[pl.ds(base0, S_PER)])

    return kern(author_h, paper_h, s_author, s_paper)


def kernel(x_author, x_paper, edge_index, supervision_edge_index,
           W_paper, b_paper, W_author, b_author):
    n_authors = x_author.shape[0]
    ei = edge_index.astype(jnp.int32)
    si = supervision_edge_index.astype(jnp.int32)
    e_author, e_paper = ei[0], ei[1]
    s_author, s_paper = si[0], si[1]

    paper_h, ph0, ph1 = _paper_linear(x_paper, W_paper,
                                      b_paper.reshape(1, -1))
    cnt = _count_sc(e_author, n_authors)
    s0, s1 = _segment_sum_sc(ph0, ph1, e_author, e_paper, n_authors)
    author_h = _author_linear(s0, s1, cnt, W_author, b_author.reshape(1, -1))
    return _score_sc(author_h, paper_h, s_author, s_paper)
